# baseline (speedup 1.0000x reference)
import functools

import jax
import jax.numpy as jnp
from jax import lax
from jax.experimental import pallas as pl
from jax.experimental.pallas import tpu as pltpu
from jax.experimental.pallas import tpu_sc as plsc

_EPS = float(jnp.finfo(jnp.float32).eps)
_PAD_FLAT = 126 * 128 + 127

_NW = 32
_LANES = 16


def _idx_body(ytT_ref, ypT_ref, flat_ref):
    xt = ytT_ref[...]
    xp = ypT_ref[...]
    C, RC = xt.shape
    mt = jnp.max(xt, axis=0, keepdims=True)
    mp = jnp.max(xp, axis=0, keepdims=True)
    iota0 = lax.broadcasted_iota(jnp.int32, (C, RC), 0)
    big = jnp.int32(127)
    t = jnp.min(jnp.where(xt == mt, iota0, big), axis=0, keepdims=True)
    p = jnp.min(jnp.where(xp == mp, iota0, big), axis=0, keepdims=True)
    flat_ref[...] = t * 128 + p


def _sc_hist_body(flat_hbm, hist_hbm, idx_v, pp_v, tp_v, part_v):
    nw_rows = flat_hbm.shape[0] // _NW
    wid = lax.axis_index("s") * 2 + lax.axis_index("c")
    base = wid * nw_rows
    pltpu.sync_copy(flat_hbm.at[pl.ds(base, nw_rows)], idx_v)

    zeros = jnp.zeros((_LANES,), jnp.float32)
    for j in range(128):
        pp_v[pl.ds(j * 16, 16)] = zeros
        tp_v[pl.ds(j * 16, 16)] = zeros

    laneoff = lax.iota(jnp.int32, _LANES) * 128
    ones = jnp.ones((_LANES,), jnp.float32)

    def body(i, carry):
        v = idx_v[pl.ds(i * _LANES, _LANES)]
        p = lax.bitwise_and(v, jnp.int32(127))
        t = lax.shift_right_logical(v, jnp.int32(7))
        addr = p + laneoff
        plsc.addupdate_scatter(pp_v, [addr], ones)
        plsc.addupdate_scatter(tp_v, [addr], ones, mask=(t == p))
        return carry

    lax.fori_loop(0, nw_rows // _LANES, body, jnp.int32(0))

    for b in range(8):
        accp = zeros
        acct = zeros
        for l in range(_LANES):
            accp = accp + pp_v[pl.ds(l * 128 + b * 16, 16)]
            acct = acct + tp_v[pl.ds(l * 128 + b * 16, 16)]
        part_v[0, pl.ds(b * 16, 16)] = accp
        part_v[1, pl.ds(b * 16, 16)] = acct
    pltpu.sync_copy(part_v, hist_hbm.at[wid])


def _fin_body(hist_ref, out_ref):
    h = hist_ref[...]
    s = jnp.sum(h, axis=0)
    pp = s[0:1, :]
    tp = s[1:2, :]
    prec = tp / (pp + _EPS)
    out_ref[...] = jnp.sum(prec, axis=1, keepdims=True) / jnp.float32(100.0)


def kernel(y_true, y_pred):
    N, C = y_true.shape
    ytT = y_true.T
    ypT = y_pred.T
    RC = 27776
    G = N // RC
    nmain = G * RC

    flat_main = pl.pallas_call(
        _idx_body,
        grid=(G,),
        in_specs=[
            pl.BlockSpec((C, RC), lambda i: (0, i)),
            pl.BlockSpec((C, RC), lambda i: (0, i)),
        ],
        out_specs=pl.BlockSpec((1, RC), lambda i: (0, i)),
        out_shape=jax.ShapeDtypeStruct((1, nmain), jnp.int32),
        compiler_params=pltpu.CompilerParams(dimension_semantics=("arbitrary",)),
    )(ytT, ypT)

    ytR = lax.slice(ytT, (0, nmain), (C, N))
    ypR = lax.slice(ypT, (0, nmain), (C, N))
    flat_rem = pl.pallas_call(
        _idx_body,
        out_shape=jax.ShapeDtypeStruct((1, N - nmain), jnp.int32),
    )(ytR, ypR)

    npad = (-N) % (_NW * _LANES)
    flat = jnp.concatenate(
        [flat_main[0], flat_rem[0],
         jnp.full((npad,), _PAD_FLAT, jnp.int32)])
    ntot = N + npad
    nw_rows = ntot // _NW

    mesh = plsc.VectorSubcoreMesh(core_axis_name="c", subcore_axis_name="s")
    hist = pl.kernel(
        _sc_hist_body,
        out_type=jax.ShapeDtypeStruct((_NW, 2, 128), jnp.float32),
        mesh=mesh,
        scratch_types=[
            pltpu.VMEM((nw_rows,), jnp.int32),
            pltpu.VMEM((128 * _LANES,), jnp.float32),
            pltpu.VMEM((128 * _LANES,), jnp.float32),
            pltpu.VMEM((2, 128), jnp.float32),
        ],
        compiler_params=pltpu.CompilerParams(needs_layout_passes=False),
    )(flat)

    out = pl.pallas_call(
        _fin_body,
        out_shape=jax.ShapeDtypeStruct((1, 1), jnp.float32),
    )(hist)
    return out[0, 0]


# speedup vs baseline: 1.4018x; 1.4018x over previous
import functools

import jax
import jax.numpy as jnp
from jax import lax
from jax.experimental import pallas as pl
from jax.experimental.pallas import tpu as pltpu

_EPS = float(jnp.finfo(jnp.float32).eps)


def _cls_body(nsteps, yta_ref, ytb_ref, ypa_ref, ypb_ref, pp_out, tp_out,
              ppa_ref, tpa_ref):
    i = pl.program_id(0)

    @pl.when(i == 0)
    def _init():
        ppa_ref[...] = jnp.zeros_like(ppa_ref)
        tpa_ref[...] = jnp.zeros_like(tpa_ref)

    for yt_ref, yp_ref in ((yta_ref, ypa_ref), (ytb_ref, ypb_ref)):
        xt = yt_ref[...]
        xp = yp_ref[...]
        mt = jnp.max(xt, axis=0, keepdims=True)
        mp = jnp.max(xp, axis=0, keepdims=True)
        eq_t = xt == mt
        eq_p = xp == mp
        ppf = eq_p.astype(jnp.float32)
        tpf = (eq_t & eq_p).astype(jnp.float32)
        C, RC = xt.shape
        g = RC // 128
        accp = ppf[:, 0:128]
        acct = tpf[:, 0:128]
        for j in range(1, g):
            accp = accp + ppf[:, j * 128:(j + 1) * 128]
            acct = acct + tpf[:, j * 128:(j + 1) * 128]
        ppa_ref[...] += accp
        tpa_ref[...] += acct

    @pl.when(i == nsteps - 1)
    def _fin():
        pp_out[...] = jnp.sum(ppa_ref[...], axis=1, keepdims=True)
        tp_out[...] = jnp.sum(tpa_ref[...], axis=1, keepdims=True)


def _fin_body(ytr_ref, ypr_ref, ppm_ref, tpm_ref, out_ref):
    xt = ytr_ref[...]
    xp = ypr_ref[...]
    mt = jnp.max(xt, axis=0, keepdims=True)
    mp = jnp.max(xp, axis=0, keepdims=True)
    eq_t = xt == mt
    eq_p = xp == mp
    pp = ppm_ref[...] + jnp.sum(eq_p.astype(jnp.float32), axis=1, keepdims=True)
    tp = tpm_ref[...] + jnp.sum((eq_t & eq_p).astype(jnp.float32), axis=1,
                                keepdims=True)
    C = pp.shape[0]
    prec = tp / (pp + _EPS)
    out_ref[...] = jnp.sum(prec, axis=0, keepdims=True) / jnp.float32(C)


def kernel(y_true, y_pred):
    N, C = y_true.shape
    ytT = y_true.T
    ypT = y_pred.T
    RC = 13824
    G = N // (2 * RC)
    nmain = 2 * G * RC
    pp_m, tp_m = pl.pallas_call(
        functools.partial(_cls_body, G),
        grid=(G,),
        in_specs=[
            pl.BlockSpec((C, RC), lambda i: (0, i)),
            pl.BlockSpec((C, RC), lambda i, _G=G: (0, i + _G)),
            pl.BlockSpec((C, RC), lambda i: (0, i)),
            pl.BlockSpec((C, RC), lambda i, _G=G: (0, i + _G)),
        ],
        out_specs=[
            pl.BlockSpec((C, 1), lambda i: (0, 0)),
            pl.BlockSpec((C, 1), lambda i: (0, 0)),
        ],
        out_shape=[
            jax.ShapeDtypeStruct((C, 1), jnp.float32),
            jax.ShapeDtypeStruct((C, 1), jnp.float32),
        ],
        scratch_shapes=[
            pltpu.VMEM((C, 128), jnp.float32),
            pltpu.VMEM((C, 128), jnp.float32),
        ],
        compiler_params=pltpu.CompilerParams(dimension_semantics=("arbitrary",)),
    )(ytT, ytT, ypT, ypT)

    ytR = lax.slice(ytT, (0, nmain), (C, N))
    ypR = lax.slice(ypT, (0, nmain), (C, N))
    out = pl.pallas_call(
        _fin_body,
        out_shape=jax.ShapeDtypeStruct((1, 1), jnp.float32),
    )(ytR, ypR, pp_m, tp_m)
    return out[0, 0]


# speedup vs baseline: 1.4357x; 1.0242x over previous
import functools

import jax
import jax.numpy as jnp
from jax import lax
from jax.experimental import pallas as pl
from jax.experimental.pallas import tpu as pltpu

_EPS = float(jnp.finfo(jnp.float32).eps)
_S = 4
_RC = 6912


def _cls_body(nsteps, *refs):
    yt_refs = refs[:_S]
    yp_refs = refs[_S:2 * _S]
    pp_out, tp_out, ppa_ref, tpa_ref = refs[2 * _S:]
    i = pl.program_id(0)

    @pl.when(i == 0)
    def _init():
        ppa_ref[...] = jnp.zeros_like(ppa_ref)
        tpa_ref[...] = jnp.zeros_like(tpa_ref)

    for yt_ref, yp_ref in zip(yt_refs, yp_refs):
        xt = yt_ref[...]
        xp = yp_ref[...]
        mt = jnp.max(xt, axis=0, keepdims=True)
        mp = jnp.max(xp, axis=0, keepdims=True)
        eq_t = xt == mt
        eq_p = xp == mp
        ppf = eq_p.astype(jnp.float32)
        tpf = (eq_t & eq_p).astype(jnp.float32)
        C, RC = xt.shape
        g = RC // 128
        accp = ppf[:, 0:128]
        acct = tpf[:, 0:128]
        for j in range(1, g):
            accp = accp + ppf[:, j * 128:(j + 1) * 128]
            acct = acct + tpf[:, j * 128:(j + 1) * 128]
        ppa_ref[...] += accp
        tpa_ref[...] += acct

    @pl.when(i == nsteps - 1)
    def _fin():
        pp_out[...] = jnp.sum(ppa_ref[...], axis=1, keepdims=True)
        tp_out[...] = jnp.sum(tpa_ref[...], axis=1, keepdims=True)


def _fin_body(ytr_ref, ypr_ref, ppm_ref, tpm_ref, out_ref):
    xt = ytr_ref[...]
    xp = ypr_ref[...]
    mt = jnp.max(xt, axis=0, keepdims=True)
    mp = jnp.max(xp, axis=0, keepdims=True)
    eq_t = xt == mt
    eq_p = xp == mp
    pp = ppm_ref[...] + jnp.sum(eq_p.astype(jnp.float32), axis=1, keepdims=True)
    tp = tpm_ref[...] + jnp.sum((eq_t & eq_p).astype(jnp.float32), axis=1,
                                keepdims=True)
    C = pp.shape[0]
    prec = tp / (pp + _EPS)
    out_ref[...] = jnp.sum(prec, axis=0, keepdims=True) / jnp.float32(C)


def _spec(C, s, G):
    return pl.BlockSpec((C, _RC), lambda i, _s=s, _G=G: (0, i + _s * _G))


def kernel(y_true, y_pred):
    N, C = y_true.shape
    ytT = y_true.T
    ypT = y_pred.T
    G = N // (_S * _RC)
    nmain = _S * G * _RC
    in_specs = ([_spec(C, s, G) for s in range(_S)] +
                [_spec(C, s, G) for s in range(_S)])
    pp_m, tp_m = pl.pallas_call(
        functools.partial(_cls_body, G),
        grid=(G,),
        in_specs=in_specs,
        out_specs=[
            pl.BlockSpec((C, 1), lambda i: (0, 0)),
            pl.BlockSpec((C, 1), lambda i: (0, 0)),
        ],
        out_shape=[
            jax.ShapeDtypeStruct((C, 1), jnp.float32),
            jax.ShapeDtypeStruct((C, 1), jnp.float32),
        ],
        scratch_shapes=[
            pltpu.VMEM((C, 128), jnp.float32),
            pltpu.VMEM((C, 128), jnp.float32),
        ],
        compiler_params=pltpu.CompilerParams(dimension_semantics=("arbitrary",)),
    )(*([ytT] * _S + [ypT] * _S))

    ytR = lax.slice(ytT, (0, nmain), (C, N))
    ypR = lax.slice(ypT, (0, nmain), (C, N))
    out = pl.pallas_call(
        _fin_body,
        out_shape=jax.ShapeDtypeStruct((1, 1), jnp.float32),
    )(ytR, ypR, pp_m, tp_m)
    return out[0, 0]


# speedup vs baseline: 1.4500x; 1.0100x over previous
import functools

import jax
import jax.numpy as jnp
from jax import lax
from jax.experimental import pallas as pl
from jax.experimental.pallas import tpu as pltpu

_EPS = float(jnp.finfo(jnp.float32).eps)
_S = 8
_RC = 3456


def _cls_body(nsteps, *refs):
    yt_refs = refs[:_S]
    yp_refs = refs[_S:2 * _S]
    pp_out, tp_out, ppa_ref, tpa_ref = refs[2 * _S:]
    i = pl.program_id(0)

    @pl.when(i == 0)
    def _init():
        ppa_ref[...] = jnp.zeros_like(ppa_ref)
        tpa_ref[...] = jnp.zeros_like(tpa_ref)

    for yt_ref, yp_ref in zip(yt_refs, yp_refs):
        xt = yt_ref[...]
        xp = yp_ref[...]
        mt = jnp.max(xt, axis=0, keepdims=True)
        mp = jnp.max(xp, axis=0, keepdims=True)
        eq_t = xt == mt
        eq_p = xp == mp
        ppf = eq_p.astype(jnp.float32)
        tpf = (eq_t & eq_p).astype(jnp.float32)
        C, RC = xt.shape
        g = RC // 128
        accp = ppf[:, 0:128]
        acct = tpf[:, 0:128]
        for j in range(1, g):
            accp = accp + ppf[:, j * 128:(j + 1) * 128]
            acct = acct + tpf[:, j * 128:(j + 1) * 128]
        ppa_ref[...] += accp
        tpa_ref[...] += acct

    @pl.when(i == nsteps - 1)
    def _fin():
        pp_out[...] = jnp.sum(ppa_ref[...], axis=1, keepdims=True)
        tp_out[...] = jnp.sum(tpa_ref[...], axis=1, keepdims=True)


def _fin_body(ytr_ref, ypr_ref, ppm_ref, tpm_ref, out_ref):
    xt = ytr_ref[...]
    xp = ypr_ref[...]
    mt = jnp.max(xt, axis=0, keepdims=True)
    mp = jnp.max(xp, axis=0, keepdims=True)
    eq_t = xt == mt
    eq_p = xp == mp
    pp = ppm_ref[...] + jnp.sum(eq_p.astype(jnp.float32), axis=1, keepdims=True)
    tp = tpm_ref[...] + jnp.sum((eq_t & eq_p).astype(jnp.float32), axis=1,
                                keepdims=True)
    C = pp.shape[0]
    prec = tp / (pp + _EPS)
    out_ref[...] = jnp.sum(prec, axis=0, keepdims=True) / jnp.float32(C)


def _spec(C, s, G):
    return pl.BlockSpec((C, _RC), lambda i, _s=s, _G=G: (0, i + _s * _G))


def kernel(y_true, y_pred):
    N, C = y_true.shape
    ytT = y_true.T
    ypT = y_pred.T
    G = N // (_S * _RC)
    nmain = _S * G * _RC
    in_specs = ([_spec(C, s, G) for s in range(_S)] +
                [_spec(C, s, G) for s in range(_S)])
    pp_m, tp_m = pl.pallas_call(
        functools.partial(_cls_body, G),
        grid=(G,),
        in_specs=in_specs,
        out_specs=[
            pl.BlockSpec((C, 1), lambda i: (0, 0)),
            pl.BlockSpec((C, 1), lambda i: (0, 0)),
        ],
        out_shape=[
            jax.ShapeDtypeStruct((C, 1), jnp.float32),
            jax.ShapeDtypeStruct((C, 1), jnp.float32),
        ],
        scratch_shapes=[
            pltpu.VMEM((C, 128), jnp.float32),
            pltpu.VMEM((C, 128), jnp.float32),
        ],
        compiler_params=pltpu.CompilerParams(dimension_semantics=("arbitrary",)),
    )(*([ytT] * _S + [ypT] * _S))

    ytR = lax.slice(ytT, (0, nmain), (C, N))
    ypR = lax.slice(ypT, (0, nmain), (C, N))
    out = pl.pallas_call(
        _fin_body,
        out_shape=jax.ShapeDtypeStruct((1, 1), jnp.float32),
    )(ytR, ypR, pp_m, tp_m)
    return out[0, 0]
